# Initial kernel scaffold; baseline (speedup 1.0000x reference)
import jax
import jax.numpy as jnp
from jax.experimental import pallas as pl


def kernel(indices, tables):
    raise NotImplementedError("write your pallas kernel here")


# speedup vs baseline: 8.0960x; 8.0960x over previous
import functools

import jax
import jax.numpy as jnp
from jax import lax
from jax.experimental import pallas as pl
from jax.experimental.pallas import tpu as pltpu
from jax.experimental.pallas import tpu_sc as plsc

B, F, V, D = 16384, 26, 8, 128
NC, NS = 2, 16
NW = NC * NS
ROWS = B * F
RPW = ROWS // NW
CH = 128
NCH = RPW // CH
LANES = 16


@functools.partial(
    pl.kernel,
    out_type=jax.ShapeDtypeStruct((ROWS, D), jnp.float32),
    mesh=plsc.VectorSubcoreMesh(core_axis_name="c", subcore_axis_name="s"),
    scratch_types=[
        pltpu.VMEM((NCH, CH), jnp.int32),
        pltpu.VMEM((NCH, CH), jnp.int32),
        pltpu.VMEM((CH, D), jnp.float32),
        pltpu.VMEM((CH, D), jnp.float32),
        pltpu.SemaphoreType.DMA,
        pltpu.SemaphoreType.DMA,
        pltpu.SemaphoreType.DMA,
        pltpu.SemaphoreType.DMA,
    ],
)
def _sc_lookup(tbl_hbm, idx_hbm, off_hbm, out_hbm,
               idx_v, off_v, buf0, buf1, g0, g1, s0, s1):
    wid = lax.axis_index("s") * NC + lax.axis_index("c")
    pltpu.sync_copy(idx_hbm.at[wid], idx_v)
    pltpu.sync_copy(off_hbm, off_v)

    def add_offsets(j, carry):
        for t in range(CH // LANES):
            sl = pl.ds(t * LANES, LANES)
            idx_v[j, sl] = idx_v[j, sl] + off_v[j, sl]
        return carry

    lax.fori_loop(0, NCH, add_offsets, 0)

    base = wid * RPW

    def start_gather(g, buf, sem):
        pltpu.async_copy(tbl_hbm.at[idx_v.at[g]], buf, sem)

    def wait_gather(g, buf, sem):
        pltpu.make_async_copy(tbl_hbm.at[idx_v.at[g]], buf, sem).wait()

    def start_scatter(g, buf, sem):
        pltpu.async_copy(buf, out_hbm.at[pl.ds(base + g * CH, CH)], sem)

    def wait_scatter(g, buf, sem):
        pltpu.make_async_copy(buf, out_hbm.at[pl.ds(base + g * CH, CH)], sem).wait()

    start_gather(0, buf0, g0)
    start_gather(1, buf1, g1)

    def body(k, carry):
        g = 2 * k
        wait_gather(g, buf0, g0)
        start_scatter(g, buf0, s0)
        wait_gather(g + 1, buf1, g1)
        start_scatter(g + 1, buf1, s1)
        wait_scatter(g, buf0, s0)
        start_gather(g + 2, buf0, g0)
        wait_scatter(g + 1, buf1, s1)
        start_gather(g + 3, buf1, g1)
        return carry

    lax.fori_loop(0, NCH // 2 - 1, body, 0)

    g = NCH - 2
    wait_gather(g, buf0, g0)
    start_scatter(g, buf0, s0)
    wait_gather(g + 1, buf1, g1)
    start_scatter(g + 1, buf1, s1)
    wait_scatter(g, buf0, s0)
    wait_scatter(g + 1, buf1, s1)


def kernel(indices, tables):
    tbl = tables.reshape(F * V, D)
    idx3 = indices.reshape(NW, NCH, CH)
    offs = ((jnp.arange(RPW, dtype=jnp.int32) % F) * V).reshape(NCH, CH)
    out = _sc_lookup(tbl, idx3, offs)
    return out.reshape(B, F * D)


# speedup vs baseline: 8.1398x; 1.0054x over previous
import functools

import jax
import jax.numpy as jnp
from jax import lax
from jax.experimental import pallas as pl
from jax.experimental.pallas import tpu as pltpu
from jax.experimental.pallas import tpu_sc as plsc

B, F, V, D = 16384, 26, 8, 128
NC, NS = 2, 16
NW = NC * NS
ROWS = B * F
RPW = ROWS // NW
CH = 128
NCH = RPW // CH
LANES = 16


@functools.partial(
    pl.kernel,
    out_type=jax.ShapeDtypeStruct((ROWS, D), jnp.float32),
    mesh=plsc.VectorSubcoreMesh(core_axis_name="c", subcore_axis_name="s"),
    scratch_types=(
        [pltpu.VMEM((NCH, CH), jnp.int32),
         pltpu.VMEM((NCH, CH), jnp.int32)]
        + [pltpu.VMEM((CH, D), jnp.float32) for _ in range(4)]
        + [pltpu.SemaphoreType.DMA for _ in range(8)]
    ),
)
def _sc_lookup(tbl_hbm, idx_hbm, off_hbm, out_hbm,
               idx_v, off_v, b0, b1, b2, b3, *sems):
    bufs = (b0, b1, b2, b3)
    gsem = sems[:4]
    ssem = sems[4:]
    NB = 4

    wid = lax.axis_index("s") * NC + lax.axis_index("c")
    pltpu.sync_copy(idx_hbm.at[wid], idx_v)
    pltpu.sync_copy(off_hbm, off_v)

    def add_offsets(j, carry):
        for t in range(CH // LANES):
            sl = pl.ds(t * LANES, LANES)
            idx_v[j, sl] = idx_v[j, sl] + off_v[j, sl]
        return carry

    lax.fori_loop(0, NCH, add_offsets, 0)

    base = wid * RPW

    def start_gather(g, p):
        pltpu.async_copy(tbl_hbm.at[idx_v.at[g]], bufs[p], gsem[p])

    def wait_gather(g, p):
        pltpu.make_async_copy(tbl_hbm.at[idx_v.at[g]], bufs[p], gsem[p]).wait()

    def start_scatter(g, p):
        pltpu.async_copy(bufs[p], out_hbm.at[pl.ds(base + g * CH, CH)], ssem[p])

    def wait_scatter(g, p):
        pltpu.make_async_copy(bufs[p], out_hbm.at[pl.ds(base + g * CH, CH)],
                              ssem[p]).wait()

    for p in range(NB):
        start_gather(p, p)

    def body(k, carry):
        g = NB * k
        for p in range(NB):
            wait_gather(g + p, p)
            start_scatter(g + p, p)
        for p in range(NB):
            wait_scatter(g + p, p)
            start_gather(g + NB + p, p)
        return carry

    lax.fori_loop(0, NCH // NB - 1, body, 0)

    g = NCH - NB
    for p in range(NB):
        wait_gather(g + p, p)
        start_scatter(g + p, p)
    for p in range(NB):
        wait_scatter(g + p, p)


def kernel(indices, tables):
    tbl = tables.reshape(F * V, D)
    idx3 = indices.reshape(NW, NCH, CH)
    offs = ((jnp.arange(RPW, dtype=jnp.int32) % F) * V).reshape(NCH, CH)
    out = _sc_lookup(tbl, idx3, offs)
    return out.reshape(B, F * D)


# speedup vs baseline: 15.1962x; 1.8669x over previous
import functools

import jax
import jax.numpy as jnp
from jax import lax
from jax.experimental import pallas as pl
from jax.experimental.pallas import tpu as pltpu
from jax.experimental.pallas import tpu_sc as plsc

B, F, V, D = 16384, 26, 8, 128
NC, NS = 2, 16
NW = NC * NS
ROWS = B * F
RPW = ROWS // NW
CH = 128
NCH = RPW // CH
LANES = 16


@functools.partial(
    pl.kernel,
    out_type=jax.ShapeDtypeStruct((ROWS, D), jnp.float32),
    mesh=plsc.VectorSubcoreMesh(core_axis_name="c", subcore_axis_name="s"),
    scratch_types=(
        [pltpu.VMEM((NCH, CH), jnp.int32),
         pltpu.VMEM((NCH, CH), jnp.int32)]
        + [pltpu.VMEM((CH, D), jnp.float32) for _ in range(4)]
        + [pltpu.VMEM_SHARED((F * V, D), jnp.float32)]
        + [pltpu.SemaphoreType.DMA for _ in range(8)]
    ),
)
def _sc_lookup(tbl_hbm, idx_hbm, off_hbm, out_hbm,
               idx_v, off_v, b0, b1, b2, b3, tbl_sh, *sems):
    bufs = (b0, b1, b2, b3)
    gsem = sems[:4]
    ssem = sems[4:]
    NB = 4

    wid = lax.axis_index("s") * NC + lax.axis_index("c")

    @pl.when(lax.axis_index("s") == 0)
    def _stage_table():
        pltpu.sync_copy(tbl_hbm, tbl_sh)

    pltpu.sync_copy(idx_hbm.at[wid], idx_v)
    pltpu.sync_copy(off_hbm, off_v)

    def add_offsets(j, carry):
        for t in range(CH // LANES):
            sl = pl.ds(t * LANES, LANES)
            idx_v[j, sl] = idx_v[j, sl] + off_v[j, sl]
        return carry

    lax.fori_loop(0, NCH, add_offsets, 0)
    plsc.subcore_barrier()

    base = wid * RPW

    def start_gather(g, p):
        pltpu.async_copy(tbl_sh.at[idx_v.at[g]], bufs[p], gsem[p])

    def wait_gather(g, p):
        pltpu.make_async_copy(tbl_sh.at[idx_v.at[g]], bufs[p], gsem[p]).wait()

    def start_scatter(g, p):
        pltpu.async_copy(bufs[p], out_hbm.at[pl.ds(base + g * CH, CH)], ssem[p])

    def wait_scatter(g, p):
        pltpu.make_async_copy(bufs[p], out_hbm.at[pl.ds(base + g * CH, CH)],
                              ssem[p]).wait()

    for p in range(NB):
        start_gather(p, p)

    def body(k, carry):
        g = NB * k
        for p in range(NB):
            wait_gather(g + p, p)
            start_scatter(g + p, p)
        for p in range(NB):
            wait_scatter(g + p, p)
            start_gather(g + NB + p, p)
        return carry

    lax.fori_loop(0, NCH // NB - 1, body, 0)

    g = NCH - NB
    for p in range(NB):
        wait_gather(g + p, p)
        start_scatter(g + p, p)
    for p in range(NB):
        wait_scatter(g + p, p)


def kernel(indices, tables):
    tbl = tables.reshape(F * V, D)
    idx3 = indices.reshape(NW, NCH, CH)
    offs = ((jnp.arange(RPW, dtype=jnp.int32) % F) * V).reshape(NCH, CH)
    out = _sc_lookup(tbl, idx3, offs)
    return out.reshape(B, F * D)


# speedup vs baseline: 34.0994x; 2.2440x over previous
import functools

import jax
import jax.numpy as jnp
from jax import lax
from jax.experimental import pallas as pl
from jax.experimental.pallas import tpu as pltpu
from jax.experimental.pallas import tpu_sc as plsc

B, F, V, D = 16384, 26, 8, 128
NC, NS = 2, 16
NW = NC * NS
ROWS = B * F
RPW = ROWS // NW
CH = 128
NCH = RPW // CH
NB = 4
NBANDS = B // 8


@functools.partial(
    pl.kernel,
    out_type=jax.ShapeDtypeStruct((NBANDS, F, 8, D), jnp.float32),
    mesh=plsc.VectorSubcoreMesh(core_axis_name="c", subcore_axis_name="s"),
    scratch_types=(
        [pltpu.VMEM((NCH, CH), jnp.int32)]
        + [pltpu.VMEM((CH, D), jnp.float32) for _ in range(NB)]
        + [pltpu.VMEM_SHARED((F * V, D), jnp.float32)]
        + [pltpu.SemaphoreType.DMA for _ in range(2 * NB)]
    ),
)
def _sc_lookup(tbl_hbm, idx_hbm, out_4d, idx_v, *rest):
    out_hbm = out_4d.reshape(ROWS, D)
    bufs = rest[:NB]
    tbl_sh = rest[NB]
    gsem = rest[NB + 1:2 * NB + 1]
    ssem = rest[2 * NB + 1:]

    wid = lax.axis_index("s") * NC + lax.axis_index("c")

    @pl.when(lax.axis_index("s") == 0)
    def _stage_table():
        pltpu.sync_copy(tbl_hbm, tbl_sh)

    pltpu.sync_copy(idx_hbm.at[wid], idx_v)
    plsc.subcore_barrier()

    base = wid * RPW

    def start_gather(g, p):
        pltpu.async_copy(tbl_sh.at[idx_v.at[g]], bufs[p], gsem[p])

    def wait_gather(g, p):
        pltpu.make_async_copy(tbl_sh.at[idx_v.at[g]], bufs[p], gsem[p]).wait()

    def start_scatter(g, p):
        pltpu.async_copy(bufs[p], out_hbm.at[pl.ds(base + g * CH, CH)], ssem[p])

    def wait_scatter(g, p):
        pltpu.make_async_copy(bufs[p], out_hbm.at[pl.ds(base + g * CH, CH)],
                              ssem[p]).wait()

    for p in range(NB):
        start_gather(p, p)

    def body(k, carry):
        g = NB * k
        for p in range(NB):
            wait_gather(g + p, p)
            start_scatter(g + p, p)
        for p in range(NB):
            wait_scatter(g + p, p)
            start_gather(g + NB + p, p)
        return carry

    lax.fori_loop(0, NCH // NB - 1, body, 0)

    g = NCH - NB
    for p in range(NB):
        wait_gather(g + p, p)
        start_scatter(g + p, p)
    for p in range(NB):
        wait_scatter(g + p, p)


def kernel(indices, tables):
    tbl = tables.reshape(F * V, D)
    flat_idx = indices + jnp.arange(F, dtype=jnp.int32)[None, :] * V
    perm_idx = flat_idx.reshape(NBANDS, 8, F).transpose(0, 2, 1)
    idx3 = perm_idx.reshape(NW, NCH, CH)
    out = _sc_lookup(tbl, idx3)
    return out.transpose(0, 2, 1, 3).reshape(B, F * D)
